# baseline (speedup 1.0000x reference)
import functools

import jax
import jax.numpy as jnp
from jax.experimental import pallas as pl
from jax.experimental.pallas import tpu as pltpu
from jax.experimental.pallas import tpu_sc as plsc

B, S, D = 4, 2048, 1024
E, TOPK, DFF = 8, 2, 4096
T = B * S
NSLOT = T * TOPK
TILE = 256
PMAX = NSLOT + E * TILE
NT = PMAX // TILE
GR = 512
SCW = 16


def _gating_body(x_ref, gw_ref, gb_ref, w1_ref, w2_ref, e1_ref, e2_ref):
    x = x_ref[...]
    logits = jax.lax.dot_general(
        x, gw_ref[...], (((1,), (0,)), ((), ())),
        precision=jax.lax.Precision.DEFAULT,
        preferred_element_type=jnp.float32) + gb_ref[...]
    m = jnp.max(logits, axis=-1, keepdims=True)
    ex = jnp.exp(logits - m)
    p = ex / jnp.sum(ex, axis=-1, keepdims=True)
    lane = jax.lax.broadcasted_iota(jnp.int32, p.shape, 1)
    m1 = jnp.max(p, axis=-1, keepdims=True)
    i1 = jnp.min(jnp.where(p >= m1, lane, E), axis=-1, keepdims=True)
    p2 = jnp.where(lane == i1, -1.0, p)
    m2 = jnp.max(p2, axis=-1, keepdims=True)
    i2 = jnp.min(jnp.where(p2 >= m2, lane, E), axis=-1, keepdims=True)
    denom = m1 + m2 + 1e-8
    w1_ref[...] = m1 / denom
    w2_ref[...] = m2 / denom
    e1_ref[...] = i1
    e2_ref[...] = i2


def _gating(x2d, gate_w, gate_b):
    grid = (T // GR,)
    return pl.pallas_call(
        _gating_body,
        grid=grid,
        in_specs=[
            pl.BlockSpec((GR, D), lambda i: (i, 0)),
            pl.BlockSpec((D, E), lambda i: (0, 0)),
            pl.BlockSpec((1, E), lambda i: (0, 0)),
        ],
        out_specs=[
            pl.BlockSpec((GR, 1), lambda i: (i, 0)),
            pl.BlockSpec((GR, 1), lambda i: (i, 0)),
            pl.BlockSpec((GR, 1), lambda i: (i, 0)),
            pl.BlockSpec((GR, 1), lambda i: (i, 0)),
        ],
        out_shape=[
            jax.ShapeDtypeStruct((T, 1), jnp.float32),
            jax.ShapeDtypeStruct((T, 1), jnp.float32),
            jax.ShapeDtypeStruct((T, 1), jnp.int32),
            jax.ShapeDtypeStruct((T, 1), jnp.int32),
        ],
    )(x2d, gate_w, gate_b.reshape(1, E))


def _route_body(e1_ref, e2_ref, d1_ref, d2_ref, te_ref):
    em = jnp.concatenate([e1_ref[...], e2_ref[...]], axis=0).T
    r_i = jax.lax.broadcasted_iota(jnp.int32, (128, 128), 0)
    c_i = jax.lax.broadcasted_iota(jnp.int32, (128, 128), 1)
    lower = (r_i > c_i).astype(jnp.float32)
    upper = (r_i < c_i).astype(jnp.float32)
    lane = jax.lax.broadcasted_iota(jnp.int32, (1, 128), 1)
    masks, totals = [], []
    for e in range(E):
        mk = (em == e).astype(jnp.float32)
        masks.append(mk)
        totals.append(jnp.sum(mk))
    starts = []
    run = jnp.float32(0.0)
    for e in range(E):
        starts.append(run)
        run = run + jnp.ceil(totals[e] / TILE) * TILE
    dest = jnp.zeros((128, 128), jnp.float32)
    for e in range(E):
        mk = masks[e]
        rank = jax.lax.dot_general(
            lower, mk, (((1,), (0,)), ((), ())),
            preferred_element_type=jnp.float32)
        cnt = jnp.sum(mk, axis=0, keepdims=True)
        excl = jax.lax.dot_general(
            cnt, upper, (((1,), (0,)), ((), ())),
            preferred_element_type=jnp.float32)
        dest = dest + mk * (starts[e] + excl + rank)
    dest_t = dest.astype(jnp.int32).T
    d1_ref[...] = dest_t[:64, :]
    d2_ref[...] = dest_t[64:, :]
    te = jnp.zeros((1, 128), jnp.int32)
    for e in range(E):
        tstart = (starts[e] / TILE).astype(jnp.int32)
        te = te + (lane >= tstart).astype(jnp.int32)
    ntiles = (run / TILE).astype(jnp.int32)
    te_ref[...] = jnp.where(lane == 127, ntiles, te - 1)


def _route(e1r, e2r):
    return pl.pallas_call(
        _route_body,
        grid=(1,),
        in_specs=[pl.BlockSpec((64, 128), lambda i: (0, 0)),
                  pl.BlockSpec((64, 128), lambda i: (0, 0))],
        out_specs=[
            pl.BlockSpec((64, 128), lambda i: (0, 0)),
            pl.BlockSpec((64, 128), lambda i: (0, 0)),
            pl.BlockSpec((1, 128), lambda i: (0, 0)),
        ],
        out_shape=[
            jax.ShapeDtypeStruct((64, 128), jnp.int32),
            jax.ShapeDtypeStruct((64, 128), jnp.int32),
            jax.ShapeDtypeStruct((1, 128), jnp.int32),
        ],
    )(e1r, e2r)


def _dispatch(x2d, d1, d2):
    mesh = plsc.VectorSubcoreMesh(core_axis_name="c", subcore_axis_name="s")

    @functools.partial(
        pl.kernel, mesh=mesh,
        out_type=jax.ShapeDtypeStruct((PMAX, D), jnp.float32),
        scratch_types=[
            pltpu.VMEM((64, D), jnp.float32),
            pltpu.VMEM((64,), jnp.int32),
            pltpu.VMEM((64,), jnp.int32),
        ],
    )
    def k(x_hbm, d1_hbm, d2_hbm, o_hbm, xrows_v, i1_v, i2_v):
        wid = jax.lax.axis_index("s") * 2 + jax.lax.axis_index("c")
        for j in range(4):
            base = (wid * 4 + j) * 64
            pltpu.sync_copy(x_hbm.at[pl.ds(base, 64)], xrows_v)
            pltpu.sync_copy(d1_hbm.at[pl.ds(base, 64)], i1_v)
            pltpu.sync_copy(d2_hbm.at[pl.ds(base, 64)], i2_v)
            pltpu.sync_copy(xrows_v, o_hbm.at[i1_v])
            pltpu.sync_copy(xrows_v, o_hbm.at[i2_v])

    return k(x2d, d1, d2)


def _ffn_body(te_ref, x_ref, w1_ref, b1_ref, w2_ref, b2_ref, o_ref):
    @pl.when(pl.program_id(0) < te_ref[127])
    def _():
        x = x_ref[...].astype(jnp.bfloat16)
        acc = jnp.zeros((TILE, D), jnp.float32)
        for c in range(2):
            lo, hi = c * 2048, (c + 1) * 2048
            h = jax.lax.dot_general(
                x, w1_ref[0, :, lo:hi], (((1,), (0,)), ((), ())),
                preferred_element_type=jnp.float32)
            h = h + b1_ref[0, :, lo:hi]
            g = 0.5 * h * (1.0 + jax.lax.erf(h * 0.7071067811865476))
            acc = acc + jax.lax.dot_general(
                g.astype(jnp.bfloat16), w2_ref[0, lo:hi, :],
                (((1,), (0,)), ((), ())),
                preferred_element_type=jnp.float32)
        o_ref[...] = acc + b2_ref[0, :, :]


def _ffn(te_arr, x_sorted, fcw, fcb, pjw, pjb):
    grid_spec = pltpu.PrefetchScalarGridSpec(
        num_scalar_prefetch=1,
        grid=(NT,),
        in_specs=[
            pl.BlockSpec((TILE, D), lambda i, te: (i, 0)),
            pl.BlockSpec((1, D, DFF), lambda i, te: (te[i], 0, 0)),
            pl.BlockSpec((1, 1, DFF), lambda i, te: (te[i], 0, 0)),
            pl.BlockSpec((1, DFF, D), lambda i, te: (te[i], 0, 0)),
            pl.BlockSpec((1, 1, D), lambda i, te: (te[i], 0, 0)),
        ],
        out_specs=pl.BlockSpec((TILE, D), lambda i, te: (i, 0)),
    )
    return pl.pallas_call(
        _ffn_body,
        grid_spec=grid_spec,
        out_shape=jax.ShapeDtypeStruct((PMAX, D), jnp.float32),
    )(te_arr, x_sorted, fcw, fcb, pjw, pjb)


def _gather2(y_sorted, d1, d2):
    mesh = plsc.VectorSubcoreMesh(core_axis_name="c", subcore_axis_name="s")

    @functools.partial(
        pl.kernel, mesh=mesh,
        out_type=[jax.ShapeDtypeStruct((T, D), jnp.float32),
                  jax.ShapeDtypeStruct((T, D), jnp.float32)],
        scratch_types=[
            pltpu.VMEM((64, D), jnp.float32),
            pltpu.VMEM((64,), jnp.int32),
        ],
    )
    def k(y_hbm, d1_hbm, d2_hbm, o1_hbm, o2_hbm, rows_v, i_v):
        wid = jax.lax.axis_index("s") * 2 + jax.lax.axis_index("c")
        for j in range(4):
            base = (wid * 4 + j) * 64
            pltpu.sync_copy(d1_hbm.at[pl.ds(base, 64)], i_v)
            pltpu.sync_copy(y_hbm.at[i_v], rows_v)
            pltpu.sync_copy(rows_v, o1_hbm.at[pl.ds(base, 64)])
            pltpu.sync_copy(d2_hbm.at[pl.ds(base, 64)], i_v)
            pltpu.sync_copy(y_hbm.at[i_v], rows_v)
            pltpu.sync_copy(rows_v, o2_hbm.at[pl.ds(base, 64)])

    return k(y_sorted, d1, d2)


def _combine_body(w1_ref, w2_ref, y1_ref, y2_ref, o_ref):
    o_ref[...] = w1_ref[...] * y1_ref[...] + w2_ref[...] * y2_ref[...]


def _combine(w1, w2, y1, y2):
    grid = (T // GR,)
    return pl.pallas_call(
        _combine_body,
        grid=grid,
        in_specs=[
            pl.BlockSpec((GR, 1), lambda i: (i, 0)),
            pl.BlockSpec((GR, 1), lambda i: (i, 0)),
            pl.BlockSpec((GR, D), lambda i: (i, 0)),
            pl.BlockSpec((GR, D), lambda i: (i, 0)),
        ],
        out_specs=pl.BlockSpec((GR, D), lambda i: (i, 0)),
        out_shape=jax.ShapeDtypeStruct((T, D), jnp.float32),
    )(w1, w2, y1, y2)


def kernel(x, gate_w, gate_b, fc_w, fc_b, proj_w, proj_b):
    x2d = x.reshape(T, D)
    w1, w2, e1, e2 = _gating(x2d, gate_w, gate_b)

    d1m, d2m, te_arr = _route(e1.reshape(64, 128), e2.reshape(64, 128))
    d1 = d1m.reshape(T)
    d2 = d2m.reshape(T)

    x_sorted = _dispatch(x2d, d1, d2)
    fc_w, te_arr = jax.lax.optimization_barrier((fc_w, te_arr))
    y_sorted = _ffn(te_arr.reshape(128), x_sorted,
                    fc_w.astype(jnp.bfloat16),
                    fc_b.reshape(E, 1, DFF),
                    proj_w.astype(jnp.bfloat16),
                    proj_b.reshape(E, 1, D))
    y1, y2 = _gather2(y_sorted, d1, d2)
    out = _combine(w1, w2, y1, y2)
    return out.reshape(B, S, D)


# speedup vs baseline: 1.0011x; 1.0011x over previous
import functools

import jax
import jax.numpy as jnp
from jax.experimental import pallas as pl
from jax.experimental.pallas import tpu as pltpu
from jax.experimental.pallas import tpu_sc as plsc

B, S, D = 4, 2048, 1024
E, TOPK, DFF = 8, 2, 4096
T = B * S
NSLOT = T * TOPK
TILE = 256
PMAX = NSLOT + E * TILE
NT = PMAX // TILE
GR = 512


def _gating_body(x_ref, gw_ref, gb_ref, w1_ref, w2_ref, e1_ref, e2_ref):
    x = x_ref[...]
    logits = jax.lax.dot_general(
        x, gw_ref[...], (((1,), (0,)), ((), ())),
        precision=jax.lax.Precision.DEFAULT,
        preferred_element_type=jnp.float32) + gb_ref[...]
    m = jnp.max(logits, axis=-1, keepdims=True)
    ex = jnp.exp(logits - m)
    p = ex / jnp.sum(ex, axis=-1, keepdims=True)
    lane = jax.lax.broadcasted_iota(jnp.int32, p.shape, 1)
    m1 = jnp.max(p, axis=-1, keepdims=True)
    i1 = jnp.min(jnp.where(p >= m1, lane, E), axis=-1, keepdims=True)
    p2 = jnp.where(lane == i1, -1.0, p)
    m2 = jnp.max(p2, axis=-1, keepdims=True)
    i2 = jnp.min(jnp.where(p2 >= m2, lane, E), axis=-1, keepdims=True)
    denom = m1 + m2 + 1e-8
    w1_ref[...] = m1 / denom
    w2_ref[...] = m2 / denom
    e1_ref[...] = i1
    e2_ref[...] = i2


def _gating(x2d, gate_w, gate_b):
    grid = (T // GR,)
    return pl.pallas_call(
        _gating_body,
        grid=grid,
        in_specs=[
            pl.BlockSpec((GR, D), lambda i: (i, 0)),
            pl.BlockSpec((D, E), lambda i: (0, 0)),
            pl.BlockSpec((1, E), lambda i: (0, 0)),
        ],
        out_specs=[
            pl.BlockSpec((GR, 1), lambda i: (i, 0)),
            pl.BlockSpec((GR, 1), lambda i: (i, 0)),
            pl.BlockSpec((GR, 1), lambda i: (i, 0)),
            pl.BlockSpec((GR, 1), lambda i: (i, 0)),
        ],
        out_shape=[
            jax.ShapeDtypeStruct((T, 1), jnp.float32),
            jax.ShapeDtypeStruct((T, 1), jnp.float32),
            jax.ShapeDtypeStruct((T, 1), jnp.int32),
            jax.ShapeDtypeStruct((T, 1), jnp.int32),
        ],
    )(x2d, gate_w, gate_b.reshape(1, E))


def _route_body(e1_ref, e2_ref, d1_ref, d2_ref, te_ref):
    em = jnp.concatenate([e1_ref[...], e2_ref[...]], axis=0).T
    r_i = jax.lax.broadcasted_iota(jnp.int32, (128, 128), 0)
    c_i = jax.lax.broadcasted_iota(jnp.int32, (128, 128), 1)
    lower = (r_i > c_i).astype(jnp.float32)
    upper = (r_i < c_i).astype(jnp.float32)
    lane = jax.lax.broadcasted_iota(jnp.int32, (1, 128), 1)
    masks, totals = [], []
    for e in range(E):
        mk = (em == e).astype(jnp.float32)
        masks.append(mk)
        totals.append(jnp.sum(mk))
    starts = []
    run = jnp.float32(0.0)
    for e in range(E):
        starts.append(run)
        run = run + jnp.ceil(totals[e] / TILE) * TILE
    dest = jnp.zeros((128, 128), jnp.float32)
    for e in range(E):
        mk = masks[e]
        rank = jax.lax.dot_general(
            lower, mk, (((1,), (0,)), ((), ())),
            preferred_element_type=jnp.float32)
        cnt = jnp.sum(mk, axis=0, keepdims=True)
        excl = jax.lax.dot_general(
            cnt, upper, (((1,), (0,)), ((), ())),
            preferred_element_type=jnp.float32)
        dest = dest + mk * (starts[e] + excl + rank)
    dest_t = dest.astype(jnp.int32).T
    d1_ref[...] = dest_t[:64, :]
    d2_ref[...] = dest_t[64:, :]
    te = jnp.zeros((1, 128), jnp.int32)
    for e in range(E):
        tstart = (starts[e] / TILE).astype(jnp.int32)
        te = te + (lane >= tstart).astype(jnp.int32)
    ntiles = (run / TILE).astype(jnp.int32)
    te_ref[...] = jnp.where(lane == 127, ntiles, te - 1)


def _route(e1r, e2r):
    return pl.pallas_call(
        _route_body,
        grid=(1,),
        in_specs=[pl.BlockSpec((64, 128), lambda i: (0, 0)),
                  pl.BlockSpec((64, 128), lambda i: (0, 0))],
        out_specs=[
            pl.BlockSpec((64, 128), lambda i: (0, 0)),
            pl.BlockSpec((64, 128), lambda i: (0, 0)),
            pl.BlockSpec((1, 128), lambda i: (0, 0)),
        ],
        out_shape=[
            jax.ShapeDtypeStruct((64, 128), jnp.int32),
            jax.ShapeDtypeStruct((64, 128), jnp.int32),
            jax.ShapeDtypeStruct((1, 128), jnp.int32),
        ],
    )(e1r, e2r)


def _dispatch(x2d, d1, d2):
    mesh = plsc.VectorSubcoreMesh(core_axis_name="c", subcore_axis_name="s")

    @functools.partial(
        pl.kernel, mesh=mesh,
        out_type=jax.ShapeDtypeStruct((PMAX, D), jnp.float32),
        scratch_types=[
            pltpu.VMEM((64, D), jnp.float32),
            pltpu.VMEM((64,), jnp.int32),
            pltpu.VMEM((64,), jnp.int32),
        ],
    )
    def k(x_hbm, d1_hbm, d2_hbm, o_hbm, xrows_v, i1_v, i2_v):
        wid = jax.lax.axis_index("s") * 2 + jax.lax.axis_index("c")
        for j in range(4):
            base = (wid * 4 + j) * 64
            pltpu.sync_copy(x_hbm.at[pl.ds(base, 64)], xrows_v)
            pltpu.sync_copy(d1_hbm.at[pl.ds(base, 64)], i1_v)
            pltpu.sync_copy(d2_hbm.at[pl.ds(base, 64)], i2_v)
            pltpu.sync_copy(xrows_v, o_hbm.at[i1_v])
            pltpu.sync_copy(xrows_v, o_hbm.at[i2_v])

    return k(x2d, d1, d2)


def _ffn_body(te_ref, x_ref, w1_ref, b1_ref, w2_ref, b2_ref, o_ref):
    @pl.when(pl.program_id(0) < te_ref[127])
    def _():
        x = x_ref[...].astype(jnp.bfloat16)
        acc = jnp.zeros((TILE, D), jnp.float32)
        for c in range(2):
            lo, hi = c * 2048, (c + 1) * 2048
            h = jax.lax.dot_general(
                x, w1_ref[0, :, lo:hi], (((1,), (0,)), ((), ())),
                preferred_element_type=jnp.float32)
            h = h + b1_ref[0, :, lo:hi]
            g = 0.5 * h * (1.0 + jax.lax.erf(h * 0.7071067811865476))
            acc = acc + jax.lax.dot_general(
                g.astype(jnp.bfloat16), w2_ref[0, lo:hi, :],
                (((1,), (0,)), ((), ())),
                preferred_element_type=jnp.float32)
        o_ref[...] = acc + b2_ref[0, :, :]


def _ffn(te_arr, x_sorted, fcw, fcb, pjw, pjb):
    grid_spec = pltpu.PrefetchScalarGridSpec(
        num_scalar_prefetch=1,
        grid=(NT,),
        in_specs=[
            pl.BlockSpec((TILE, D), lambda i, te: (i, 0)),
            pl.BlockSpec((1, D, DFF), lambda i, te: (te[i], 0, 0)),
            pl.BlockSpec((1, 1, DFF), lambda i, te: (te[i], 0, 0)),
            pl.BlockSpec((1, DFF, D), lambda i, te: (te[i], 0, 0)),
            pl.BlockSpec((1, 1, D), lambda i, te: (te[i], 0, 0)),
        ],
        out_specs=pl.BlockSpec((TILE, D), lambda i, te: (i, 0)),
    )
    return pl.pallas_call(
        _ffn_body,
        grid_spec=grid_spec,
        out_shape=jax.ShapeDtypeStruct((PMAX, D), jnp.float32),
    )(te_arr, x_sorted, fcw, fcb, pjw, pjb)


def _gather2(y_sorted, d1, d2):
    mesh = plsc.VectorSubcoreMesh(core_axis_name="c", subcore_axis_name="s")

    @functools.partial(
        pl.kernel, mesh=mesh,
        out_type=[jax.ShapeDtypeStruct((T, D), jnp.float32),
                  jax.ShapeDtypeStruct((T, D), jnp.float32)],
        scratch_types=[
            pltpu.VMEM((64, D), jnp.float32),
            pltpu.VMEM((64,), jnp.int32),
        ],
    )
    def k(y_hbm, d1_hbm, d2_hbm, o1_hbm, o2_hbm, rows_v, i_v):
        wid = jax.lax.axis_index("s") * 2 + jax.lax.axis_index("c")
        for j in range(4):
            base = (wid * 4 + j) * 64
            pltpu.sync_copy(d1_hbm.at[pl.ds(base, 64)], i_v)
            pltpu.sync_copy(y_hbm.at[i_v], rows_v)
            pltpu.sync_copy(rows_v, o1_hbm.at[pl.ds(base, 64)])
            pltpu.sync_copy(d2_hbm.at[pl.ds(base, 64)], i_v)
            pltpu.sync_copy(y_hbm.at[i_v], rows_v)
            pltpu.sync_copy(rows_v, o2_hbm.at[pl.ds(base, 64)])

    return k(y_sorted, d1, d2)


def _combine_body(w1_ref, w2_ref, y1_ref, y2_ref, o_ref):
    o_ref[...] = w1_ref[...] * y1_ref[...] + w2_ref[...] * y2_ref[...]


def _combine(w1, w2, y1, y2):
    grid = (T // GR,)
    return pl.pallas_call(
        _combine_body,
        grid=grid,
        in_specs=[
            pl.BlockSpec((GR, 1), lambda i: (i, 0)),
            pl.BlockSpec((GR, 1), lambda i: (i, 0)),
            pl.BlockSpec((GR, D), lambda i: (i, 0)),
            pl.BlockSpec((GR, D), lambda i: (i, 0)),
        ],
        out_specs=pl.BlockSpec((GR, D), lambda i: (i, 0)),
        out_shape=jax.ShapeDtypeStruct((T, D), jnp.float32),
    )(w1, w2, y1, y2)


def kernel(x, gate_w, gate_b, fc_w, fc_b, proj_w, proj_b):
    x2d = x.reshape(T, D)
    w1, w2, e1, e2 = _gating(x2d, gate_w, gate_b)

    d1m, d2m, te_arr = _route(e1.reshape(64, 128), e2.reshape(64, 128))
    d1 = d1m.reshape(T)
    d2 = d2m.reshape(T)

    x_sorted = _dispatch(x2d, d1, d2)
    fc_w, te_arr = jax.lax.optimization_barrier((fc_w, te_arr))
    y_sorted = _ffn(te_arr.reshape(128), x_sorted,
                    fc_w.astype(jnp.bfloat16),
                    fc_b.reshape(E, 1, DFF),
                    proj_w.astype(jnp.bfloat16),
                    proj_b.reshape(E, 1, D))
    y1, y2 = _gather2(y_sorted, d1, d2)
    out = _combine(w1, w2, y1, y2)
    return out.reshape(B, S, D)
